# Initial kernel scaffold; baseline (speedup 1.0000x reference)
import jax
import jax.numpy as jnp
from jax.experimental import pallas as pl


def kernel(x, edge_index, W1, b1, W2, b2, W3, b3, W4, b4, g1, be1, g2, be2, g3, be3, Wm, bm):
    raise NotImplementedError("write your pallas kernel here")


# speedup vs baseline: 2.2306x; 2.2306x over previous
import functools

import jax
import jax.numpy as jnp
from jax import lax
from jax.experimental import pallas as pl
from jax.experimental.pallas import tpu as pltpu
from jax.experimental.pallas import tpu_sc as plsc

N = 50000
E = 800000
HID = 128
NP = 51200
EP = 819200
BN = 2048
GRID = NP // BN
NC = 2
NS = 16
L = 16
IS = 512
CHH = 1024
CHS = 1024
RPT = NP // NS
ZR = 800

_MESH = plsc.VectorSubcoreMesh(core_axis_name="c", subcore_axis_name="s")


def _make_prop(edge_split, ch):
    ni = ch // IS
    npass = 1 if edge_split else 4
    cpt = EP // (NC if edge_split else 1) // NS // ch
    out_rows = (NC if edge_split else 1) * NP
    out_t = jax.ShapeDtypeStruct((out_rows, 8, 1, 16), jnp.float32)

    @functools.partial(
        pl.kernel, out_type=out_t, mesh=_MESH,
        compiler_params=pltpu.CompilerParams(use_tc_tiling_on_sc=False),
        scratch_types=[
            pltpu.VMEM((2, ni, IS), jnp.int32),
            pltpu.VMEM((2, ni, IS), jnp.int32),
            pltpu.VMEM((2, ch, 1, 16), jnp.float32),
            pltpu.VMEM((ZR, 1, 16), jnp.float32),
            pltpu.VMEM_SHARED((NP, 1, 16), jnp.float32),
            pltpu.SemaphoreType.DMA,
            pltpu.SemaphoreType.DMA,
            pltpu.SemaphoreType.DMA,
            pltpu.SemaphoreType.DMA,
        ])
    def prop(data, src_r, dst_r, out, sidx, didx, rows, zbuf, acc,
             gsem0, gsem1, ssem0, ssem1):
        c = lax.axis_index("c")
        s = lax.axis_index("s")
        zv = jnp.zeros((L,), jnp.float32)
        gsems = (gsem0, gsem1)
        ssems = (ssem0, ssem1)

        @pl.loop(0, ZR)
        def _(r):
            zbuf[r, 0, pl.ds(0, L)] = zv

        for p in range(npass):
            if edge_split:
                erow0 = c * (EP // NC // ch)
                obase = c * NP
                b = 0
            else:
                b = c * npass + p
                erow0 = 0
                obase = 0

            def _stage(gc, pb):
                if edge_split:
                    pltpu.sync_copy(src_r.at[gc], sidx.at[pb])
                else:
                    pltpu.sync_copy(src_r.at[b, gc], sidx.at[pb])
                pltpu.sync_copy(dst_r.at[gc], didx.at[pb])

            def _gathers(pb, fire):
                for j in range(ni):
                    a, d = data.at[sidx.at[pb, j]], rows.at[pb, pl.ds(j * IS, IS)]
                    if fire:
                        pltpu.async_copy(a, d, gsems[pb])
                    else:
                        pltpu.make_async_copy(a, d, gsems[pb]).wait()

            def _scatters(pb, fire):
                for j in range(ni):
                    a, d = rows.at[pb, pl.ds(j * IS, IS)], acc.at[didx.at[pb, j]]
                    if fire:
                        pltpu.async_copy(a, d, ssems[pb], add=True)
                    else:
                        pltpu.make_async_copy(a, d, ssems[pb]).wait()

            @pl.loop(0, RPT, step=ZR)
            def _(r):
                pltpu.sync_copy(zbuf, acc.at[pl.ds(s * RPT + r, ZR)])

            plsc.subcore_barrier()

            _stage(erow0 + s * cpt, 0)
            _gathers(0, fire=True)

            @pl.loop(0, cpt)
            def _(g):
                for par in range(2):
                    @pl.when(g % 2 == par)
                    def _():
                        cur, nxt = par, 1 - par

                        @pl.when(g >= 1)
                        def _():
                            _scatters(nxt, fire=False)

                        @pl.when(g + 1 < cpt)
                        def _():
                            _stage(erow0 + s * cpt + g + 1, nxt)
                            _gathers(nxt, fire=True)

                        _gathers(cur, fire=False)
                        _scatters(cur, fire=True)

            _scatters((cpt - 1) % 2, fire=False)
            plsc.subcore_barrier()
            pltpu.sync_copy(acc.at[pl.ds(s * RPT, RPT)],
                            out.at[pl.ds(obase + s * RPT, RPT), b])

    return prop


_proph = _make_prop(edge_split=False, ch=CHH)
_prop16 = _make_prop(edge_split=True, ch=CHS)


_ROW = pl.BlockSpec((BN, HID), lambda i: (i, 0))
_DIS = pl.BlockSpec((BN, 1), lambda i: (i, 0))
_VEC = pl.BlockSpec((1, HID), lambda i: (0, 0))
_PAIR = pl.BlockSpec((2, BN, HID), lambda i: (0, i, 0))
_WMAT = pl.BlockSpec((3, HID, HID), lambda i: (0, 0, 0))

_F_ROW = jax.ShapeDtypeStruct((NP, HID), jnp.float32)
_F_VEC = jax.ShapeDtypeStruct((1, HID), jnp.float32)


def _dis_body(degp, x128, dis_o, a0_o):
    deg = degp[0, :, 0:1] + degp[1, :, 0:1]
    dis = jnp.where(deg > 0, lax.rsqrt(deg), 0.0)
    dis_o[...] = dis
    a0_o[...] = -dis * x128[...]


def _dis_call(degp, x128):
    return pl.pallas_call(
        _dis_body,
        grid=(GRID,),
        in_specs=[_PAIR, _ROW],
        out_specs=[_DIS, _ROW],
        out_shape=[jax.ShapeDtypeStruct((NP, 1), jnp.float32), _F_ROW],
    )(degp, x128)


def _zpad(t):
    return jnp.concatenate(
        [t, jnp.zeros((BN, HID - 16), jnp.float32)], axis=1)


def _sum16_body(sp, dis, tx1_o, a1_o):
    t = dis[...] * (sp[0, :, 0:16] + sp[1, :, 0:16])
    tx1_o[...] = _zpad(t)
    a1_o[...] = _zpad(-dis[...] * t)


def _sum16_call(sp, dis):
    return pl.pallas_call(
        _sum16_body,
        grid=(GRID,),
        in_specs=[_PAIR, _DIS],
        out_specs=[_ROW, _ROW],
        out_shape=[_F_ROW, _F_ROW],
    )(sp, dis)


def _stats_accum(i, a, ssum_o, ssq_o):
    rows = lax.broadcasted_iota(jnp.int32, (BN, 1), 0) + i * BN
    m = (rows < N).astype(jnp.float32)
    am = a * m
    ps = jnp.sum(am, axis=0, keepdims=True)
    pq = jnp.sum(am * am, axis=0, keepdims=True)

    @pl.when(i == 0)
    def _():
        ssum_o[...] = ps
        ssq_o[...] = pq

    @pl.when(i > 0)
    def _():
        ssum_o[...] += ps
        ssq_o[...] += pq


def _l1fin_body(x128, tx1, sp1, dis, w, bias, act_o, ssum_o, ssq_o):
    i = pl.program_id(0)
    x0 = x128[...]
    t2 = _zpad(2.0 * dis[...] * (sp1[0, :, 0:16] + sp1[1, :, 0:16])) - x0
    out = jnp.dot(x0, w[0], preferred_element_type=jnp.float32)
    out += jnp.dot(tx1[...], w[1], preferred_element_type=jnp.float32)
    out += jnp.dot(t2, w[2], preferred_element_type=jnp.float32)
    out += bias[...]
    a = jnp.where(out >= 0, out, 0.01 * out)
    act_o[...] = a
    _stats_accum(i, a, ssum_o, ssq_o)


def _l1fin_call(x128, tx1, sp1, dis, w, bias):
    return pl.pallas_call(
        _l1fin_body,
        grid=(GRID,),
        in_specs=[_ROW, _ROW, _PAIR, _DIS, _WMAT, _VEC],
        out_specs=[_ROW, _VEC, _VEC],
        out_shape=[_F_ROW, _F_VEC, _F_VEC],
    )(x128, tx1, sp1, dis, w, bias)


def _hblk_body(act, sc, tc, dis, h_o, a0_o):
    h = act[...] * sc[...] + tc[...]
    h_o[...] = h
    a0_o[...] = -dis[...] * h


def _hblk_call(act, sc, tc, dis):
    return pl.pallas_call(
        _hblk_body,
        grid=(GRID,),
        in_specs=[_ROW, _VEC, _VEC, _DIS],
        out_specs=[_ROW, _ROW],
        out_shape=[_F_ROW, _F_ROW],
    )(act, sc, tc, dis)


def _a1_body(s0, dis, a1_o):
    d = dis[...]
    a1_o[...] = -(d * d) * s0[...]


def _a1_call(s0, dis):
    return pl.pallas_call(
        _a1_body,
        grid=(GRID,),
        in_specs=[_ROW, _DIS],
        out_specs=_ROW,
        out_shape=_F_ROW,
    )(s0, dis)


def _conv_acc(h, s0, s1, dis, w, bias):
    d = dis[...]
    t0 = h[...]
    t1 = d * s0[...]
    t2 = 2.0 * (d * s1[...]) - t0
    acc = jnp.zeros((BN, HID), jnp.float32) + bias[...]
    acc += jnp.dot(t0, w[0], preferred_element_type=jnp.float32)
    acc += jnp.dot(t1, w[1], preferred_element_type=jnp.float32)
    acc += jnp.dot(t2, w[2], preferred_element_type=jnp.float32)
    return acc


def _make_mid_body(leaky):
    def body(h, s0, s1, dis, w, bias, act_o, ssum_o, ssq_o):
        i = pl.program_id(0)
        acc = _conv_acc(h, s0, s1, dis, w, bias)
        if leaky:
            a = jnp.where(acc >= 0, acc, 0.01 * acc)
        else:
            a = jnp.maximum(acc, 0.0)
        act_o[...] = a
        _stats_accum(i, a, ssum_o, ssq_o)
    return body


def _mid_call(h, s0, s1, dis, w, bias, leaky):
    return pl.pallas_call(
        _make_mid_body(leaky),
        grid=(GRID,),
        in_specs=[_ROW, _ROW, _ROW, _DIS, _WMAT, _VEC],
        out_specs=[_ROW, _VEC, _VEC],
        out_shape=[_F_ROW, _F_VEC, _F_VEC],
    )(h, s0, s1, dis, w, bias)


def _fin_body(h, s0, s1, dis, w, bias, wmt, bm, y_o):
    acc = _conv_acc(h, s0, s1, dis, w, bias)
    nrm = jnp.sqrt(jnp.sum(acc * acc, axis=1, keepdims=True))
    hn = acc / jnp.maximum(nrm, 1e-12)
    cols = [jnp.sum(hn * wmt[c:c + 1, :], axis=1, keepdims=True)
            for c in range(3)]
    y_o[...] = jnp.concatenate(cols, axis=1) + bm[...]


def _fin_call(h, s0, s1, dis, w, bias, wmt, bm):
    return pl.pallas_call(
        _fin_body,
        grid=(GRID,),
        in_specs=[_ROW, _ROW, _ROW, _DIS, _WMAT, _VEC,
                  pl.BlockSpec((3, HID), lambda i: (0, 0)),
                  pl.BlockSpec((1, 3), lambda i: (0, 0))],
        out_specs=pl.BlockSpec((BN, 3), lambda i: (i, 0)),
        out_shape=jax.ShapeDtypeStruct((NP, 3), jnp.float32),
    )(h, s0, s1, dis, w, bias, wmt, bm)


def _bn_fold(ssum, ssq, g, be, eps=1e-5):
    m = ssum[0] / N
    v = ssq[0] / N - m * m
    s = g / jnp.sqrt(v + eps)
    t = be - m * s
    return s.reshape(1, HID), t.reshape(1, HID)


def _scv(a):
    return a.reshape(a.shape[0] * 8, 1, 16)


def kernel(x, edge_index, W1, b1, W2, b2, W3, b3, W4, b4,
           g1, be1, g2, be2, g3, be3, Wm, bm):
    src = edge_index[0]
    dst = edge_index[1]
    pad = jnp.full((EP - E,), N, jnp.int32)
    src_p = jnp.concatenate([src, pad])
    dst_p = jnp.concatenate([dst, pad])
    src_g = src_p.reshape(EP // CHS, CHS // IS, IS)
    dst_s = dst_p.reshape(EP // CHS, CHS // IS, IS)
    dst_h = dst_p.reshape(EP // CHH, CHH // IS, IS)
    src8 = src_p * 8
    src0 = src8.reshape(EP // CHS, CHS // IS, IS)
    src8b = (src8[None, :] + jnp.arange(8, dtype=jnp.int32)[:, None]
             ).reshape(8, EP // CHH, CHH // IS, IS)

    x128 = jnp.zeros((NP, HID), jnp.float32).at[:N, :3].set(x)
    ones = jnp.ones((NP, HID), jnp.float32)

    degp = _prop16(_scv(ones), src_g, src_g).reshape(2, NP, HID)
    dis, a0 = _dis_call(degp, x128)

    sp0 = _prop16(_scv(a0), src0, dst_s).reshape(2, NP, HID)
    tx1, a1 = _sum16_call(sp0, dis)
    sp1 = _prop16(_scv(a1), src0, dst_s).reshape(2, NP, HID)
    w1p = jnp.pad(W1, ((0, 0), (0, HID - 3), (0, 0)))
    act, ssum, ssq = _l1fin_call(x128, tx1, sp1, dis, w1p, b1.reshape(1, HID))
    s, t = _bn_fold(ssum, ssq, g1, be1)
    h, a0h = _hblk_call(act, s, t, dis)

    for (W, bb, g, be, leaky) in ((W2, b2, g2, be2, True),
                                  (W3, b3, g3, be3, False)):
        s0 = _proph(_scv(a0h), src8b, dst_h).reshape(NP, HID)
        a1h = _a1_call(s0, dis)
        s1 = _proph(_scv(a1h), src8b, dst_h).reshape(NP, HID)
        act, ssum, ssq = _mid_call(h, s0, s1, dis, W, bb.reshape(1, HID),
                                   leaky)
        s, t = _bn_fold(ssum, ssq, g, be)
        h, a0h = _hblk_call(act, s, t, dis)

    s0 = _proph(_scv(a0h), src8b, dst_h).reshape(NP, HID)
    a1h = _a1_call(s0, dis)
    s1 = _proph(_scv(a1h), src8b, dst_h).reshape(NP, HID)
    y = _fin_call(h, s0, s1, dis, W4, b4.reshape(1, HID), Wm.T,
                  bm.reshape(1, 3))
    return y[:N]


# speedup vs baseline: 2.8946x; 1.2977x over previous
import functools

import jax
import jax.numpy as jnp
from jax import lax
from jax.experimental import pallas as pl
from jax.experimental.pallas import tpu as pltpu
from jax.experimental.pallas import tpu_sc as plsc

N = 50000
E = 800000
HID = 128
NP = 51200
EP = 819200
BN = 2048
GRID = NP // BN
NC = 2
NS = 16
L = 16
IS = 512
CHH = 1024
CHS = 1024
RPT = NP // NS
ZR = 800

_MESH = plsc.VectorSubcoreMesh(core_axis_name="c", subcore_axis_name="s")


def _make_prop(edge_split, ch):
    ni = ch // IS
    npass = 1 if edge_split else 4
    cpt = EP // (NC if edge_split else 1) // NS // ch
    out_rows = (NC if edge_split else 1) * NP
    out_t = jax.ShapeDtypeStruct((out_rows, HID), jnp.float32)

    @functools.partial(
        pl.kernel, out_type=out_t, mesh=_MESH,
        compiler_params=pltpu.CompilerParams(use_tc_tiling_on_sc=False),
        scratch_types=[
            pltpu.VMEM((2, ni, IS), jnp.int32),
            pltpu.VMEM((2, ni, IS), jnp.int32),
            pltpu.VMEM((2, ch, 1, 16), jnp.float32),
            pltpu.VMEM((ZR, 1, 16), jnp.float32),
            pltpu.VMEM_SHARED((NP, 1, 16), jnp.float32),
            pltpu.SemaphoreType.DMA,
            pltpu.SemaphoreType.DMA,
            pltpu.SemaphoreType.DMA,
            pltpu.SemaphoreType.DMA,
        ])
    def prop(data, src_r, dst_r, out, sidx, didx, rows, zbuf, acc,
             gsem0, gsem1, ssem0, ssem1):
        c = lax.axis_index("c")
        s = lax.axis_index("s")
        zv = jnp.zeros((L,), jnp.float32)
        gsems = (gsem0, gsem1)
        ssems = (ssem0, ssem1)

        @pl.loop(0, ZR)
        def _(r):
            zbuf[r, 0, pl.ds(0, L)] = zv

        for p in range(npass):
            if edge_split:
                erow0 = c * (EP // NC // ch)
                obase = c * NP
                b = 0
            else:
                b = c * npass + p
                erow0 = 0
                obase = 0

            def _stage(gc, pb):
                if edge_split:
                    pltpu.sync_copy(src_r.at[gc], sidx.at[pb])
                else:
                    pltpu.sync_copy(src_r.at[b, gc], sidx.at[pb])
                pltpu.sync_copy(dst_r.at[gc], didx.at[pb])

            def _gathers(pb, fire):
                for j in range(ni):
                    a = data.at[sidx.at[pb, j]]
                    d = rows.at[pb, pl.ds(j * IS, IS)]
                    if fire:
                        pltpu.async_copy(a, d, gsems[pb])
                    else:
                        pltpu.make_async_copy(a, d, gsems[pb]).wait()

            def _scatters(pb, fire):
                for j in range(ni):
                    a, d = rows.at[pb, pl.ds(j * IS, IS)], acc.at[didx.at[pb, j]]
                    if fire:
                        pltpu.async_copy(a, d, ssems[pb], add=True)
                    else:
                        pltpu.make_async_copy(a, d, ssems[pb]).wait()

            @pl.loop(0, RPT, step=ZR)
            def _(r):
                pltpu.sync_copy(zbuf, acc.at[pl.ds(s * RPT + r, ZR)])

            plsc.subcore_barrier()

            _stage(erow0 + s * cpt, 0)
            _gathers(0, fire=True)

            @pl.loop(0, cpt)
            def _(g):
                for par in range(2):
                    @pl.when(g % 2 == par)
                    def _():
                        cur, nxt = par, 1 - par

                        @pl.when(g >= 1)
                        def _():
                            _scatters(nxt, fire=False)

                        @pl.when(g + 1 < cpt)
                        def _():
                            _stage(erow0 + s * cpt + g + 1, nxt)
                            _gathers(nxt, fire=True)

                        _gathers(cur, fire=False)
                        _scatters(cur, fire=True)

            _scatters((cpt - 1) % 2, fire=False)
            plsc.subcore_barrier()
            pltpu.sync_copy(acc.at[pl.ds(s * RPT, RPT), 0],
                            out.at[pl.ds(obase + s * RPT, RPT),
                                   pl.ds(b * 16, 16)])

    return prop


_proph = _make_prop(edge_split=False, ch=CHH)
_prop16 = _make_prop(edge_split=True, ch=CHS)


_ROW = pl.BlockSpec((BN, HID), lambda i: (i, 0))
_DIS = pl.BlockSpec((BN, 1), lambda i: (i, 0))
_VEC = pl.BlockSpec((1, HID), lambda i: (0, 0))
_PAIR = pl.BlockSpec((2, BN, HID), lambda i: (0, i, 0))
_WMAT = pl.BlockSpec((3, HID, HID), lambda i: (0, 0, 0))

_F_ROW = jax.ShapeDtypeStruct((NP, HID), jnp.float32)
_F_VEC = jax.ShapeDtypeStruct((1, HID), jnp.float32)


def _dis_body(degp, x128, dis_o, a0_o):
    deg = degp[0, :, 0:1] + degp[1, :, 0:1]
    dis = jnp.where(deg > 0, lax.rsqrt(deg), 0.0)
    dis_o[...] = dis
    a0_o[...] = -dis * x128[...]


def _dis_call(degp, x128):
    return pl.pallas_call(
        _dis_body,
        grid=(GRID,),
        in_specs=[_PAIR, _ROW],
        out_specs=[_DIS, _ROW],
        out_shape=[jax.ShapeDtypeStruct((NP, 1), jnp.float32), _F_ROW],
    )(degp, x128)


def _zpad(t):
    return jnp.concatenate(
        [t, jnp.zeros((BN, HID - 16), jnp.float32)], axis=1)


def _sum16_body(sp, dis, tx1_o, a1_o):
    t = dis[...] * (sp[0, :, 0:16] + sp[1, :, 0:16])
    tx1_o[...] = _zpad(t)
    a1_o[...] = _zpad(-dis[...] * t)


def _sum16_call(sp, dis):
    return pl.pallas_call(
        _sum16_body,
        grid=(GRID,),
        in_specs=[_PAIR, _DIS],
        out_specs=[_ROW, _ROW],
        out_shape=[_F_ROW, _F_ROW],
    )(sp, dis)


def _stats_accum(i, a, ssum_o, ssq_o):
    rows = lax.broadcasted_iota(jnp.int32, (BN, 1), 0) + i * BN
    m = (rows < N).astype(jnp.float32)
    am = a * m
    ps = jnp.sum(am, axis=0, keepdims=True)
    pq = jnp.sum(am * am, axis=0, keepdims=True)

    @pl.when(i == 0)
    def _():
        ssum_o[...] = ps
        ssq_o[...] = pq

    @pl.when(i > 0)
    def _():
        ssum_o[...] += ps
        ssq_o[...] += pq


def _l1fin_body(x128, tx1, sp1, dis, w, bias, act_o, ssum_o, ssq_o):
    i = pl.program_id(0)
    x0 = x128[...]
    t2 = _zpad(2.0 * dis[...] * (sp1[0, :, 0:16] + sp1[1, :, 0:16])) - x0
    out = jnp.dot(x0, w[0], preferred_element_type=jnp.float32)
    out += jnp.dot(tx1[...], w[1], preferred_element_type=jnp.float32)
    out += jnp.dot(t2, w[2], preferred_element_type=jnp.float32)
    out += bias[...]
    a = jnp.where(out >= 0, out, 0.01 * out)
    act_o[...] = a
    _stats_accum(i, a, ssum_o, ssq_o)


def _l1fin_call(x128, tx1, sp1, dis, w, bias):
    return pl.pallas_call(
        _l1fin_body,
        grid=(GRID,),
        in_specs=[_ROW, _ROW, _PAIR, _DIS, _WMAT, _VEC],
        out_specs=[_ROW, _VEC, _VEC],
        out_shape=[_F_ROW, _F_VEC, _F_VEC],
    )(x128, tx1, sp1, dis, w, bias)


def _hblk_body(act, sc, tc, dis, h_o, a0_o):
    h = act[...] * sc[...] + tc[...]
    h_o[...] = h
    a0_o[...] = -dis[...] * h


def _hblk_call(act, sc, tc, dis):
    return pl.pallas_call(
        _hblk_body,
        grid=(GRID,),
        in_specs=[_ROW, _VEC, _VEC, _DIS],
        out_specs=[_ROW, _ROW],
        out_shape=[_F_ROW, _F_ROW],
    )(act, sc, tc, dis)


def _a1_body(s0, dis, a1_o):
    d = dis[...]
    a1_o[...] = -(d * d) * s0[...]


def _a1_call(s0, dis):
    return pl.pallas_call(
        _a1_body,
        grid=(GRID,),
        in_specs=[_ROW, _DIS],
        out_specs=_ROW,
        out_shape=_F_ROW,
    )(s0, dis)


def _conv_acc(h, s0, s1, dis, w, bias):
    d = dis[...]
    t0 = h[...]
    t1 = d * s0[...]
    t2 = 2.0 * (d * s1[...]) - t0
    acc = jnp.zeros((BN, HID), jnp.float32) + bias[...]
    acc += jnp.dot(t0, w[0], preferred_element_type=jnp.float32)
    acc += jnp.dot(t1, w[1], preferred_element_type=jnp.float32)
    acc += jnp.dot(t2, w[2], preferred_element_type=jnp.float32)
    return acc


def _make_mid_body(leaky):
    def body(h, s0, s1, dis, w, bias, act_o, ssum_o, ssq_o):
        i = pl.program_id(0)
        acc = _conv_acc(h, s0, s1, dis, w, bias)
        if leaky:
            a = jnp.where(acc >= 0, acc, 0.01 * acc)
        else:
            a = jnp.maximum(acc, 0.0)
        act_o[...] = a
        _stats_accum(i, a, ssum_o, ssq_o)
    return body


def _mid_call(h, s0, s1, dis, w, bias, leaky):
    return pl.pallas_call(
        _make_mid_body(leaky),
        grid=(GRID,),
        in_specs=[_ROW, _ROW, _ROW, _DIS, _WMAT, _VEC],
        out_specs=[_ROW, _VEC, _VEC],
        out_shape=[_F_ROW, _F_VEC, _F_VEC],
    )(h, s0, s1, dis, w, bias)


def _fin_body(h, s0, s1, dis, w, bias, wmt, bm, y_o):
    acc = _conv_acc(h, s0, s1, dis, w, bias)
    nrm = jnp.sqrt(jnp.sum(acc * acc, axis=1, keepdims=True))
    hn = acc / jnp.maximum(nrm, 1e-12)
    cols = [jnp.sum(hn * wmt[c:c + 1, :], axis=1, keepdims=True)
            for c in range(3)]
    y_o[...] = jnp.concatenate(cols, axis=1) + bm[...]


def _fin_call(h, s0, s1, dis, w, bias, wmt, bm):
    return pl.pallas_call(
        _fin_body,
        grid=(GRID,),
        in_specs=[_ROW, _ROW, _ROW, _DIS, _WMAT, _VEC,
                  pl.BlockSpec((3, HID), lambda i: (0, 0)),
                  pl.BlockSpec((1, 3), lambda i: (0, 0))],
        out_specs=pl.BlockSpec((BN, 3), lambda i: (i, 0)),
        out_shape=jax.ShapeDtypeStruct((NP, 3), jnp.float32),
    )(h, s0, s1, dis, w, bias, wmt, bm)


def _bn_fold(ssum, ssq, g, be, eps=1e-5):
    m = ssum[0] / N
    v = ssq[0] / N - m * m
    s = g / jnp.sqrt(v + eps)
    t = be - m * s
    return s.reshape(1, HID), t.reshape(1, HID)


def kernel(x, edge_index, W1, b1, W2, b2, W3, b3, W4, b4,
           g1, be1, g2, be2, g3, be3, Wm, bm):
    src = edge_index[0]
    dst = edge_index[1]
    pad = jnp.full((EP - E,), N, jnp.int32)
    src_p = jnp.concatenate([src, pad])
    dst_p = jnp.concatenate([dst, pad])
    src_g = src_p.reshape(EP // CHS, CHS // IS, IS)
    dst_s = dst_p.reshape(EP // CHS, CHS // IS, IS)
    src8 = src_p * 8
    src0 = src8.reshape(EP // CHS, CHS // IS, IS)
    src8b = (src8[None, :] + jnp.arange(8, dtype=jnp.int32)[:, None]
             ).reshape(8, EP // CHH, CHH // IS, IS)

    x128 = jnp.zeros((NP, HID), jnp.float32).at[:N, :3].set(x)
    ones = jnp.ones((NP, HID), jnp.float32)

    ones8 = ones.reshape(NP * 8, 1, 16)
    degp = _prop16(ones8, src_g, src_g).reshape(2, NP, HID)
    dis, a0 = _dis_call(degp, x128)

    sp0 = _prop16(a0.reshape(NP * 8, 1, 16), src0, dst_s).reshape(2, NP, HID)
    tx1, a1 = _sum16_call(sp0, dis)
    sp1 = _prop16(a1.reshape(NP * 8, 1, 16), src0, dst_s).reshape(2, NP, HID)
    w1p = jnp.pad(W1, ((0, 0), (0, HID - 3), (0, 0)))
    act, ssum, ssq = _l1fin_call(x128, tx1, sp1, dis, w1p, b1.reshape(1, HID))
    s, t = _bn_fold(ssum, ssq, g1, be1)
    h, a0h = _hblk_call(act, s, t, dis)

    for (W, bb, g, be, leaky) in ((W2, b2, g2, be2, True),
                                  (W3, b3, g3, be3, False)):
        s0 = _proph(a0h.reshape(NP * 8, 1, 16), src8b, dst_s)
        a1h = _a1_call(s0, dis)
        s1 = _proph(a1h.reshape(NP * 8, 1, 16), src8b, dst_s)
        act, ssum, ssq = _mid_call(h, s0, s1, dis, W, bb.reshape(1, HID),
                                   leaky)
        s, t = _bn_fold(ssum, ssq, g, be)
        h, a0h = _hblk_call(act, s, t, dis)

    s0 = _proph(a0h.reshape(NP * 8, 1, 16), src8b, dst_s)
    a1h = _a1_call(s0, dis)
    s1 = _proph(a1h.reshape(NP * 8, 1, 16), src8b, dst_s)
    y = _fin_call(h, s0, s1, dis, W4, b4.reshape(1, HID), Wm.T,
                  bm.reshape(1, 3))
    return y[:N]
